# baseline (speedup 1.0000x reference)
import functools

import jax
import jax.numpy as jnp
import numpy as np
from jax import lax
from jax.experimental import pallas as pl
from jax.experimental.pallas import tpu as pltpu
from jax.experimental.pallas import tpu_sc as plsc

F32 = jnp.float32


def _matmul_bias_body(x_ref, w_ref, b_ref, o_ref):
    o_ref[...] = (
        jnp.dot(x_ref[...], w_ref[...], preferred_element_type=F32) + b_ref[...]
    )


def _matmul_bias(x, w, b, blk):
    n, k = x.shape
    _, m = w.shape
    grid = n // blk
    return pl.pallas_call(
        _matmul_bias_body,
        grid=(grid,),
        in_specs=[
            pl.BlockSpec((blk, k), lambda i: (i, 0)),
            pl.BlockSpec((k, m), lambda i: (0, 0)),
            pl.BlockSpec((1, m), lambda i: (0, 0)),
        ],
        out_specs=pl.BlockSpec((blk, m), lambda i: (i, 0)),
        out_shape=jax.ShapeDtypeStruct((n, m), F32),
    )(x, w, b.reshape(1, m))


def _mlp_body(h_ref, a0_ref, a1_ref, w1_ref, b1_ref, w2_ref, b2_ref, o_ref,
              *, final):
    h_in = h_ref[...] + a0_ref[...] + a1_ref[...]
    mid = jnp.maximum(
        jnp.dot(h_in, w1_ref[...], preferred_element_type=F32) + b1_ref[...], 0.0
    )
    out = jnp.dot(mid, w2_ref[...], preferred_element_type=F32) + b2_ref[...]
    if not final:
        out = jnp.maximum(out, 0.0)
    o_ref[...] = out


def _gin_mlp(h, agg2, w1, b1, w2, b2, final, blk):
    n, d = h.shape
    dh = w1.shape[1]
    nblk = n // blk
    return pl.pallas_call(
        functools.partial(_mlp_body, final=final),
        grid=(nblk,),
        in_specs=[
            pl.BlockSpec((blk, d), lambda i: (i, 0)),
            pl.BlockSpec((blk, d), lambda i: (i, 0)),
            pl.BlockSpec((blk, d), lambda i, _n=nblk: (i + _n, 0)),
            pl.BlockSpec((d, dh), lambda i: (0, 0)),
            pl.BlockSpec((1, dh), lambda i: (0, 0)),
            pl.BlockSpec((dh, d), lambda i: (0, 0)),
            pl.BlockSpec((1, d), lambda i: (0, 0)),
        ],
        out_specs=pl.BlockSpec((blk, d), lambda i: (i, 0)),
        out_shape=jax.ShapeDtypeStruct((n, d), F32),
    )(h, agg2, agg2, w1, b1.reshape(1, dh), w2, b2.reshape(1, d))


def _mish_tc(x):
    sp = jnp.maximum(x, 0.0) + jnp.log1p(jnp.exp(-jnp.abs(x)))
    return x * jnp.tanh(sp)


def _pool_body(node_ref, batch_ref, gpw_ref, gpb_ref, nfwb_ref,
               grep_ref, gout_ref, gtnp_ref, acc_ref, cnt_ref, *, ngraph, nsteps):
    i = pl.program_id(0)

    @pl.when(i == 0)
    def _():
        acc_ref[...] = jnp.zeros_like(acc_ref)
        cnt_ref[...] = jnp.zeros_like(cnt_ref)

    bblk = batch_ref[0, 0, :]
    onehot = (
        lax.broadcasted_iota(jnp.int32, (ngraph, bblk.shape[0]), 0)
        == bblk[None, :]
    ).astype(F32)
    acc_ref[...] += jnp.dot(onehot, node_ref[...], preferred_element_type=F32)
    cnt_ref[...] += jnp.sum(onehot, axis=1, keepdims=True)

    @pl.when(i == nsteps - 1)
    def _():
        grep = acc_ref[...] / jnp.maximum(cnt_ref[...], 1.0)
        grep_ref[...] = grep
        gout_ref[...] = (
            jnp.dot(grep, gpw_ref[...], preferred_element_type=F32) + gpb_ref[...]
        )
        gtnp_ref[...] = jnp.dot(grep, nfwb_ref[...], preferred_element_type=F32)


def _pool(node_rep, batch3, gp_w, gp_b, nf_w_bot, blk):
    n, d = node_rep.shape
    g = gp_w.shape[0] if gp_w.shape[0] != d else 256
    g = 256
    nsteps = n // blk
    return pl.pallas_call(
        functools.partial(_pool_body, ngraph=g, nsteps=nsteps),
        grid=(nsteps,),
        in_specs=[
            pl.BlockSpec((blk, d), lambda i: (i, 0)),
            pl.BlockSpec((1, 1, blk), lambda i: (i, 0, 0)),
            pl.BlockSpec((d, d), lambda i: (0, 0)),
            pl.BlockSpec((1, d), lambda i: (0, 0)),
            pl.BlockSpec((d, d), lambda i: (0, 0)),
        ],
        out_specs=[
            pl.BlockSpec((g, d), lambda i: (0, 0)),
            pl.BlockSpec((g, d), lambda i: (0, 0)),
            pl.BlockSpec((g, d), lambda i: (0, 0)),
        ],
        out_shape=[
            jax.ShapeDtypeStruct((g, d), F32),
            jax.ShapeDtypeStruct((g, d), F32),
            jax.ShapeDtypeStruct((g, d), F32),
        ],
        scratch_shapes=[
            pltpu.VMEM((g, d), F32),
            pltpu.VMEM((g, 1), F32),
        ],
    )(node_rep, batch3, gp_w, gp_b.reshape(1, d), nf_w_bot)


def _heads_body(node_ref, batch_ref, gtnp_ref, nfwt_ref, nfb_ref,
                efwt_ref, efwb_ref, efb_ref,
                n2_ref, se_ref, a_ref, bt_ref, *, ngraph):
    bblk = batch_ref[0, 0, :]
    onehot = (
        lax.broadcasted_iota(jnp.int32, (bblk.shape[0], ngraph), 1)
        == bblk[:, None]
    ).astype(F32)
    gtn = jnp.dot(onehot, gtnp_ref[...], preferred_element_type=F32)
    pre = (
        jnp.dot(node_ref[...], nfwt_ref[...], preferred_element_type=F32)
        + gtn + nfb_ref[...]
    )
    n2 = _mish_tc(pre)
    n2_ref[...] = n2
    a = jnp.dot(n2, efwt_ref[...], preferred_element_type=F32)
    bt = jnp.dot(n2, efwb_ref[...], preferred_element_type=F32) + efb_ref[...]
    a_ref[...] = a
    bt_ref[...] = bt
    se_ref[...] = _mish_tc(a + bt)


def _heads(node_rep, batch3, gtnp, nf_w_top, nf_b, ef_w_top, ef_w_bot, ef_b, blk):
    n, d = node_rep.shape
    g = gtnp.shape[0]
    nsteps = n // blk
    outs = pl.pallas_call(
        functools.partial(_heads_body, ngraph=g),
        grid=(nsteps,),
        in_specs=[
            pl.BlockSpec((blk, d), lambda i: (i, 0)),
            pl.BlockSpec((1, 1, blk), lambda i: (i, 0, 0)),
            pl.BlockSpec((g, d), lambda i: (0, 0)),
            pl.BlockSpec((d, d), lambda i: (0, 0)),
            pl.BlockSpec((1, d), lambda i: (0, 0)),
            pl.BlockSpec((d, d), lambda i: (0, 0)),
            pl.BlockSpec((d, d), lambda i: (0, 0)),
            pl.BlockSpec((1, d), lambda i: (0, 0)),
        ],
        out_specs=[pl.BlockSpec((blk, d), lambda i: (i, 0))] * 4,
        out_shape=[jax.ShapeDtypeStruct((n, d), F32)] * 4,
    )(node_rep, batch3, gtnp, nf_w_top, nf_b.reshape(1, d),
      ef_w_top, ef_w_bot, ef_b.reshape(1, d))
    return outs


_EBLK = 128


_MBLK = 128


def _msgpass_body(h_hbm, e_hbm, ei_hbm, zeros_hbm, out_hbm,
                  agg, idx_v, rows_v, e_v, gsem,
                  *, n_nodes, n_pad, n_edges, emb):
    c = lax.axis_index("c")
    s = lax.axis_index("s")
    blk_core = (n_edges // 2) // _MBLK
    bpt = blk_core // 16
    rows_per_tile = n_pad // 16

    zlo = s * rows_per_tile
    pltpu.sync_copy(zeros_hbm.at[pl.ds(zlo, rows_per_tile)],
                    agg.at[pl.ds(zlo, rows_per_tile)])
    plsc.subcore_barrier()

    ebase = (c * 16 + s) * bpt * _MBLK

    def blk_body(i, _):
        e0 = ebase + i * _MBLK
        pltpu.sync_copy(ei_hbm.at[:, pl.ds(e0, _MBLK)], idx_v)
        cp = pltpu.async_copy(h_hbm.at[idx_v.at[0]], rows_v, gsem)
        pltpu.sync_copy(e_hbm.at[pl.ds(e0, _MBLK)], e_v)
        cp.wait()

        def row_body(r, _):
            for j in range(emb // 16):
                sl = pl.ds(j * 16, 16)
                e_v[r, sl] = jnp.maximum(rows_v[r, sl] + e_v[r, sl], 0.0)
            return 0

        lax.fori_loop(0, _MBLK, row_body, 0)
        pltpu.sync_copy(e_v, agg.at[idx_v.at[1]], add=True)
        return 0

    lax.fori_loop(0, bpt, blk_body, 0)

    plsc.subcore_barrier()
    last_rows = n_nodes - 15 * rows_per_tile

    @pl.when(s < 15)
    def _():
        pltpu.sync_copy(agg.at[pl.ds(zlo, rows_per_tile)],
                        out_hbm.at[pl.ds(c * n_nodes + zlo, rows_per_tile)])

    @pl.when(s == 15)
    def _():
        pltpu.sync_copy(agg.at[pl.ds(zlo, last_rows)],
                        out_hbm.at[pl.ds(c * n_nodes + zlo, last_rows)])


def _msgpass(h, e, ei2, zeros):
    n, emb = h.shape
    n_pad = zeros.shape[0]
    n_edges = e.shape[0]
    mesh = plsc.VectorSubcoreMesh(core_axis_name="c", subcore_axis_name="s")
    body = functools.partial(_msgpass_body, n_nodes=n, n_pad=n_pad,
                             n_edges=n_edges, emb=emb)
    return pl.kernel(
        body,
        out_type=jax.ShapeDtypeStruct((2 * n, emb), F32),
        mesh=mesh,
        scratch_types=[
            pltpu.VMEM_SHARED((n_pad, emb), F32),
            pltpu.VMEM((2, _MBLK), jnp.int32),
            pltpu.VMEM((_MBLK, emb), F32),
            pltpu.VMEM((_MBLK, emb), F32),
            pltpu.SemaphoreType.DMA,
        ],
    )(h, e, ei2, zeros)


def _edgegather_body(a_hbm, b_hbm, src_hbm, dst_hbm, ag_hbm, bg_hbm,
                     src_v, dst_v, arow_v, brow_v, sem_a, sem_b,
                     *, n_edges, emb):
    c = lax.axis_index("c")
    s = lax.axis_index("s")
    w = s * 2 + c
    nblk = n_edges // _EBLK
    ntile = (nblk - w + 31) // 32

    def blk_body(i, _):
        b = w + i * 32
        e0 = b * _EBLK
        pltpu.sync_copy(src_hbm.at[pl.ds(e0, _EBLK)], src_v)
        pltpu.sync_copy(dst_hbm.at[pl.ds(e0, _EBLK)], dst_v)
        ca = pltpu.async_copy(a_hbm.at[src_v], arow_v, sem_a)
        cb = pltpu.async_copy(b_hbm.at[dst_v], brow_v, sem_b)
        ca.wait()
        cb.wait()
        pltpu.sync_copy(arow_v, ag_hbm.at[pl.ds(e0, _EBLK)])
        pltpu.sync_copy(brow_v, bg_hbm.at[pl.ds(e0, _EBLK)])
        return 0

    lax.fori_loop(0, ntile, blk_body, 0)


def _edgegather(a, bt, src, dst):
    n, emb = a.shape
    n_edges = src.shape[0]
    mesh = plsc.VectorSubcoreMesh(core_axis_name="c", subcore_axis_name="s")
    body = functools.partial(_edgegather_body, n_edges=n_edges, emb=emb)
    return pl.kernel(
        body,
        out_type=[jax.ShapeDtypeStruct((n_edges, emb), F32)] * 2,
        mesh=mesh,
        scratch_types=[
            pltpu.VMEM((_EBLK,), jnp.int32),
            pltpu.VMEM((_EBLK,), jnp.int32),
            pltpu.VMEM((_EBLK, emb), F32),
            pltpu.VMEM((_EBLK, emb), F32),
            pltpu.SemaphoreType.DMA,
            pltpu.SemaphoreType.DMA,
        ],
    )(a, bt, src, dst)


def _edgemish_body(ag_ref, bg_ref, o_ref):
    x = ag_ref[...] + bg_ref[...]
    m = _mish_tc(x)
    o_ref[...] = 0.5 * (m[:, 0, :] + m[:, 1, :])


def _edgemish(ag, bg, n_edges, blk):
    emb = ag.shape[1]
    half = n_edges // 2
    ag2 = ag.reshape(ag.shape[0] // 2, 2, emb)
    bg2 = bg.reshape(bg.shape[0] // 2, 2, emb)
    grid = half // blk
    return pl.pallas_call(
        _edgemish_body,
        grid=(grid,),
        in_specs=[
            pl.BlockSpec((blk, 2, emb), lambda i: (i, 0, 0)),
            pl.BlockSpec((blk, 2, emb), lambda i: (i, 0, 0)),
        ],
        out_specs=pl.BlockSpec((blk, emb), lambda i: (i, 0)),
        out_shape=jax.ShapeDtypeStruct((half, emb), F32),
    )(ag2, bg2)


def kernel(x, edge_index, edge_attr, batch, params):
    n, _ = x.shape
    emb = params['atom_w'].shape[1]
    num_layers = params['edge_w'].shape[0]
    src = edge_index[0]
    dst = edge_index[1]
    n_edges = src.shape[0]
    nblk = 1000
    batch3 = batch.reshape(n // nblk, 1, nblk)
    rpt = ((n + 15) // 16 + 7) // 8 * 8
    n_pad = 16 * rpt
    zeros = jnp.zeros((n_pad, emb), F32)

    unit = 2 * 16 * 2 * _EBLK
    e_pad = ((n_edges + unit - 1) // unit) * unit
    pad = e_pad - n_edges
    ea_pad = jnp.pad(edge_attr, ((0, pad), (0, 0)))
    src_p = jnp.pad(src, (0, pad))
    dst_mp = jnp.pad(dst, (0, pad), constant_values=n_pad - 1)
    dst_eg = jnp.pad(dst, (0, pad))
    ei2 = jnp.stack([src_p, dst_mp])

    h = _matmul_bias(x, params['atom_w'], params['atom_b'], nblk)
    for l in range(num_layers):
        e = _matmul_bias(ea_pad, params['edge_w'][l], params['edge_b'][l], 4096)
        agg2 = _msgpass(h, e, ei2, zeros)
        h = _gin_mlp(h, agg2, params['mlp_w1'][l], params['mlp_b1'][l],
                     params['mlp_w2'][l], params['mlp_b2'][l],
                     final=(l == num_layers - 1), blk=nblk)

    nf_w = params['nf_w']
    ef_w = params['ef_w']
    _, graph_out, gtnp = _pool(h, batch3, params['gp_w'], params['gp_b'],
                               nf_w[emb:], nblk)
    n2, se, a, bt = _heads(h, batch3, gtnp, nf_w[:emb], params['nf_b'],
                           ef_w[:emb], ef_w[emb:], params['ef_b'], nblk)
    ag, bg = _edgegather(a, bt, src_p, dst_eg)
    edge_rep = _edgemish(ag, bg, n_edges, 2000)
    return (n2, se, edge_rep, graph_out)


# speedup vs baseline: 1.0441x; 1.0441x over previous
import functools

import jax
import jax.numpy as jnp
import numpy as np
from jax import lax
from jax.experimental import pallas as pl
from jax.experimental.pallas import tpu as pltpu
from jax.experimental.pallas import tpu_sc as plsc

F32 = jnp.float32


def _matmul_bias_body(x_ref, w_ref, b_ref, o_ref):
    o_ref[...] = (
        jnp.dot(x_ref[...], w_ref[...], preferred_element_type=F32) + b_ref[...]
    )


def _matmul_bias(x, w, b, blk):
    n, k = x.shape
    _, m = w.shape
    grid = n // blk
    return pl.pallas_call(
        _matmul_bias_body,
        grid=(grid,),
        in_specs=[
            pl.BlockSpec((blk, k), lambda i: (i, 0)),
            pl.BlockSpec((k, m), lambda i: (0, 0)),
            pl.BlockSpec((1, m), lambda i: (0, 0)),
        ],
        out_specs=pl.BlockSpec((blk, m), lambda i: (i, 0)),
        out_shape=jax.ShapeDtypeStruct((n, m), F32),
    )(x, w, b.reshape(1, m))


def _mlp_body(h_ref, a0_ref, a1_ref, w1_ref, b1_ref, w2_ref, b2_ref, o_ref,
              *, final):
    h_in = h_ref[...] + a0_ref[...] + a1_ref[...]
    mid = jnp.maximum(
        jnp.dot(h_in, w1_ref[...], preferred_element_type=F32) + b1_ref[...], 0.0
    )
    out = jnp.dot(mid, w2_ref[...], preferred_element_type=F32) + b2_ref[...]
    if not final:
        out = jnp.maximum(out, 0.0)
    o_ref[...] = out


def _gin_mlp(h, agg2, w1, b1, w2, b2, final, blk):
    n, d = h.shape
    dh = w1.shape[1]
    nblk = n // blk
    return pl.pallas_call(
        functools.partial(_mlp_body, final=final),
        grid=(nblk,),
        in_specs=[
            pl.BlockSpec((blk, d), lambda i: (i, 0)),
            pl.BlockSpec((blk, d), lambda i: (i, 0)),
            pl.BlockSpec((blk, d), lambda i, _n=nblk: (i + _n, 0)),
            pl.BlockSpec((d, dh), lambda i: (0, 0)),
            pl.BlockSpec((1, dh), lambda i: (0, 0)),
            pl.BlockSpec((dh, d), lambda i: (0, 0)),
            pl.BlockSpec((1, d), lambda i: (0, 0)),
        ],
        out_specs=pl.BlockSpec((blk, d), lambda i: (i, 0)),
        out_shape=jax.ShapeDtypeStruct((n, d), F32),
    )(h, agg2, agg2, w1, b1.reshape(1, dh), w2, b2.reshape(1, d))


def _mish_tc(x):
    sp = jnp.maximum(x, 0.0) + jnp.log1p(jnp.exp(-jnp.abs(x)))
    return x * jnp.tanh(sp)


def _pool_body(node_ref, batch_ref, gpw_ref, gpb_ref, nfwb_ref,
               grep_ref, gout_ref, gtnp_ref, acc_ref, cnt_ref, *, ngraph, nsteps):
    i = pl.program_id(0)

    @pl.when(i == 0)
    def _():
        acc_ref[...] = jnp.zeros_like(acc_ref)
        cnt_ref[...] = jnp.zeros_like(cnt_ref)

    bblk = batch_ref[0, 0, :]
    onehot = (
        lax.broadcasted_iota(jnp.int32, (ngraph, bblk.shape[0]), 0)
        == bblk[None, :]
    ).astype(F32)
    acc_ref[...] += jnp.dot(onehot, node_ref[...], preferred_element_type=F32)
    cnt_ref[...] += jnp.sum(onehot, axis=1, keepdims=True)

    @pl.when(i == nsteps - 1)
    def _():
        grep = acc_ref[...] / jnp.maximum(cnt_ref[...], 1.0)
        grep_ref[...] = grep
        gout_ref[...] = (
            jnp.dot(grep, gpw_ref[...], preferred_element_type=F32) + gpb_ref[...]
        )
        gtnp_ref[...] = jnp.dot(grep, nfwb_ref[...], preferred_element_type=F32)


def _pool(node_rep, batch3, gp_w, gp_b, nf_w_bot, blk):
    n, d = node_rep.shape
    g = gp_w.shape[0] if gp_w.shape[0] != d else 256
    g = 256
    nsteps = n // blk
    return pl.pallas_call(
        functools.partial(_pool_body, ngraph=g, nsteps=nsteps),
        grid=(nsteps,),
        in_specs=[
            pl.BlockSpec((blk, d), lambda i: (i, 0)),
            pl.BlockSpec((1, 1, blk), lambda i: (i, 0, 0)),
            pl.BlockSpec((d, d), lambda i: (0, 0)),
            pl.BlockSpec((1, d), lambda i: (0, 0)),
            pl.BlockSpec((d, d), lambda i: (0, 0)),
        ],
        out_specs=[
            pl.BlockSpec((g, d), lambda i: (0, 0)),
            pl.BlockSpec((g, d), lambda i: (0, 0)),
            pl.BlockSpec((g, d), lambda i: (0, 0)),
        ],
        out_shape=[
            jax.ShapeDtypeStruct((g, d), F32),
            jax.ShapeDtypeStruct((g, d), F32),
            jax.ShapeDtypeStruct((g, d), F32),
        ],
        scratch_shapes=[
            pltpu.VMEM((g, d), F32),
            pltpu.VMEM((g, 1), F32),
        ],
    )(node_rep, batch3, gp_w, gp_b.reshape(1, d), nf_w_bot)


def _heads_body(node_ref, batch_ref, gtnp_ref, nfwt_ref, nfb_ref,
                efwt_ref, efwb_ref, efb_ref,
                n2_ref, se_ref, a_ref, bt_ref, *, ngraph):
    bblk = batch_ref[0, 0, :]
    onehot = (
        lax.broadcasted_iota(jnp.int32, (bblk.shape[0], ngraph), 1)
        == bblk[:, None]
    ).astype(F32)
    gtn = jnp.dot(onehot, gtnp_ref[...], preferred_element_type=F32)
    pre = (
        jnp.dot(node_ref[...], nfwt_ref[...], preferred_element_type=F32)
        + gtn + nfb_ref[...]
    )
    n2 = _mish_tc(pre)
    n2_ref[...] = n2
    a = jnp.dot(n2, efwt_ref[...], preferred_element_type=F32)
    bt = jnp.dot(n2, efwb_ref[...], preferred_element_type=F32) + efb_ref[...]
    a_ref[...] = a
    bt_ref[...] = bt
    se_ref[...] = _mish_tc(a + bt)


def _heads(node_rep, batch3, gtnp, nf_w_top, nf_b, ef_w_top, ef_w_bot, ef_b, blk):
    n, d = node_rep.shape
    g = gtnp.shape[0]
    nsteps = n // blk
    outs = pl.pallas_call(
        functools.partial(_heads_body, ngraph=g),
        grid=(nsteps,),
        in_specs=[
            pl.BlockSpec((blk, d), lambda i: (i, 0)),
            pl.BlockSpec((1, 1, blk), lambda i: (i, 0, 0)),
            pl.BlockSpec((g, d), lambda i: (0, 0)),
            pl.BlockSpec((d, d), lambda i: (0, 0)),
            pl.BlockSpec((1, d), lambda i: (0, 0)),
            pl.BlockSpec((d, d), lambda i: (0, 0)),
            pl.BlockSpec((d, d), lambda i: (0, 0)),
            pl.BlockSpec((1, d), lambda i: (0, 0)),
        ],
        out_specs=[pl.BlockSpec((blk, d), lambda i: (i, 0))] * 4,
        out_shape=[jax.ShapeDtypeStruct((n, d), F32)] * 4,
    )(node_rep, batch3, gtnp, nf_w_top, nf_b.reshape(1, d),
      ef_w_top, ef_w_bot, ef_b.reshape(1, d))
    return outs


_EBLK = 128


_MBLK = 128


def _msgpass_body(h_hbm, e_hbm, src_hbm, dst_hbm, zeros_hbm, out_hbm,
                  agg, src_v, dst_v, rows_v, e_v, gsem,
                  *, n_nodes, n_pad, n_edges, emb):
    c = lax.axis_index("c")
    s = lax.axis_index("s")
    blk_core = (n_edges // 2) // _MBLK
    bpt = blk_core // 16
    rows_per_tile = n_pad // 16

    zlo = s * rows_per_tile
    pltpu.sync_copy(zeros_hbm.at[pl.ds(zlo, rows_per_tile)],
                    agg.at[pl.ds(zlo, rows_per_tile)])
    plsc.subcore_barrier()

    base_e = c * (n_edges // 2)

    def blk_body(i, _):
        b = s + i * 16
        e0 = base_e + b * _MBLK
        pltpu.sync_copy(src_hbm.at[pl.ds(e0, _MBLK)], src_v)
        pltpu.sync_copy(dst_hbm.at[pl.ds(e0, _MBLK)], dst_v)
        cp = pltpu.async_copy(h_hbm.at[src_v], rows_v, gsem)
        pltpu.sync_copy(e_hbm.at[pl.ds(e0, _MBLK)], e_v)
        cp.wait()

        def row_body(r, _):
            for j in range(emb // 16):
                sl = pl.ds(j * 16, 16)
                e_v[r, sl] = jnp.maximum(rows_v[r, sl] + e_v[r, sl], 0.0)
            return 0

        lax.fori_loop(0, _MBLK, row_body, 0)
        pltpu.sync_copy(e_v, agg.at[dst_v], add=True)
        return 0

    lax.fori_loop(0, bpt, blk_body, 0)

    plsc.subcore_barrier()
    last_rows = n_nodes - 15 * rows_per_tile

    @pl.when(s < 15)
    def _():
        pltpu.sync_copy(agg.at[pl.ds(zlo, rows_per_tile)],
                        out_hbm.at[pl.ds(c * n_nodes + zlo, rows_per_tile)])

    @pl.when(s == 15)
    def _():
        pltpu.sync_copy(agg.at[pl.ds(zlo, last_rows)],
                        out_hbm.at[pl.ds(c * n_nodes + zlo, last_rows)])


def _msgpass(h, e, src, dst, zeros):
    n, emb = h.shape
    n_pad = zeros.shape[0]
    n_edges = e.shape[0]
    mesh = plsc.VectorSubcoreMesh(core_axis_name="c", subcore_axis_name="s")
    body = functools.partial(_msgpass_body, n_nodes=n, n_pad=n_pad,
                             n_edges=n_edges, emb=emb)
    return pl.kernel(
        body,
        out_type=jax.ShapeDtypeStruct((2 * n, emb), F32),
        mesh=mesh,
        scratch_types=[
            pltpu.VMEM_SHARED((n_pad, emb), F32),
            pltpu.VMEM((_MBLK,), jnp.int32),
            pltpu.VMEM((_MBLK,), jnp.int32),
            pltpu.VMEM((_MBLK, emb), F32),
            pltpu.VMEM((_MBLK, emb), F32),
            pltpu.SemaphoreType.DMA,
        ],
    )(h, e, src, dst, zeros)


def _edgegather_body(a_hbm, b_hbm, src_hbm, dst_hbm, ag_hbm, bg_hbm,
                     src_v, dst_v, arow_v, brow_v, sem_a, sem_b,
                     *, n_edges, emb):
    c = lax.axis_index("c")
    s = lax.axis_index("s")
    w = s * 2 + c
    nblk = n_edges // _EBLK
    ntile = (nblk - w + 31) // 32

    def blk_body(i, _):
        b = w + i * 32
        e0 = b * _EBLK
        pltpu.sync_copy(src_hbm.at[pl.ds(e0, _EBLK)], src_v)
        pltpu.sync_copy(dst_hbm.at[pl.ds(e0, _EBLK)], dst_v)
        ca = pltpu.async_copy(a_hbm.at[src_v], arow_v, sem_a)
        cb = pltpu.async_copy(b_hbm.at[dst_v], brow_v, sem_b)
        ca.wait()
        cb.wait()
        pltpu.sync_copy(arow_v, ag_hbm.at[pl.ds(e0, _EBLK)])
        pltpu.sync_copy(brow_v, bg_hbm.at[pl.ds(e0, _EBLK)])
        return 0

    lax.fori_loop(0, ntile, blk_body, 0)


def _edgegather(a, bt, src, dst):
    n, emb = a.shape
    n_edges = src.shape[0]
    mesh = plsc.VectorSubcoreMesh(core_axis_name="c", subcore_axis_name="s")
    body = functools.partial(_edgegather_body, n_edges=n_edges, emb=emb)
    return pl.kernel(
        body,
        out_type=[jax.ShapeDtypeStruct((n_edges, emb), F32)] * 2,
        mesh=mesh,
        scratch_types=[
            pltpu.VMEM((_EBLK,), jnp.int32),
            pltpu.VMEM((_EBLK,), jnp.int32),
            pltpu.VMEM((_EBLK, emb), F32),
            pltpu.VMEM((_EBLK, emb), F32),
            pltpu.SemaphoreType.DMA,
            pltpu.SemaphoreType.DMA,
        ],
    )(a, bt, src, dst)


def _edgemish_body(ag_ref, bg_ref, o_ref):
    x = ag_ref[...] + bg_ref[...]
    m = _mish_tc(x)
    o_ref[...] = 0.5 * (m[:, 0, :] + m[:, 1, :])


def _edgemish(ag, bg, n_edges, blk):
    emb = ag.shape[1]
    half = n_edges // 2
    ag2 = ag.reshape(ag.shape[0] // 2, 2, emb)
    bg2 = bg.reshape(bg.shape[0] // 2, 2, emb)
    grid = half // blk
    return pl.pallas_call(
        _edgemish_body,
        grid=(grid,),
        in_specs=[
            pl.BlockSpec((blk, 2, emb), lambda i: (i, 0, 0)),
            pl.BlockSpec((blk, 2, emb), lambda i: (i, 0, 0)),
        ],
        out_specs=pl.BlockSpec((blk, emb), lambda i: (i, 0)),
        out_shape=jax.ShapeDtypeStruct((half, emb), F32),
    )(ag2, bg2)


def kernel(x, edge_index, edge_attr, batch, params):
    n, _ = x.shape
    emb = params['atom_w'].shape[1]
    num_layers = params['edge_w'].shape[0]
    src = edge_index[0]
    dst = edge_index[1]
    n_edges = src.shape[0]
    nblk = 1000
    batch3 = batch.reshape(n // nblk, 1, nblk)
    rpt = ((n + 15) // 16 + 7) // 8 * 8
    n_pad = 16 * rpt
    zeros = jnp.zeros((n_pad, emb), F32)

    unit = 2 * 16 * 2 * _EBLK
    e_pad = ((n_edges + unit - 1) // unit) * unit
    pad = e_pad - n_edges
    ea_pad = jnp.pad(edge_attr, ((0, pad), (0, 0)))
    src_p = jnp.pad(src, (0, pad))
    dst_mp = jnp.pad(dst, (0, pad), constant_values=n_pad - 1)
    dst_eg = jnp.pad(dst, (0, pad))

    h = _matmul_bias(x, params['atom_w'], params['atom_b'], nblk)
    for l in range(num_layers):
        e = _matmul_bias(ea_pad, params['edge_w'][l], params['edge_b'][l], 4096)
        agg2 = _msgpass(h, e, src_p, dst_mp, zeros)
        h = _gin_mlp(h, agg2, params['mlp_w1'][l], params['mlp_b1'][l],
                     params['mlp_w2'][l], params['mlp_b2'][l],
                     final=(l == num_layers - 1), blk=nblk)

    nf_w = params['nf_w']
    ef_w = params['ef_w']
    _, graph_out, gtnp = _pool(h, batch3, params['gp_w'], params['gp_b'],
                               nf_w[emb:], nblk)
    n2, se, a, bt = _heads(h, batch3, gtnp, nf_w[:emb], params['nf_b'],
                           ef_w[:emb], ef_w[emb:], params['ef_b'], nblk)
    ag, bg = _edgegather(a, bt, src_p, dst_eg)
    edge_rep = _edgemish(ag, bg, n_edges, 2000)
    return (n2, se, edge_rep, graph_out)


# speedup vs baseline: 1.0447x; 1.0006x over previous
import functools

import jax
import jax.numpy as jnp
import numpy as np
from jax import lax
from jax.experimental import pallas as pl
from jax.experimental.pallas import tpu as pltpu
from jax.experimental.pallas import tpu_sc as plsc

F32 = jnp.float32


def _matmul_bias_body(x_ref, w_ref, b_ref, o_ref):
    o_ref[...] = (
        jnp.dot(x_ref[...], w_ref[...], preferred_element_type=F32) + b_ref[...]
    )


def _matmul_bias(x, w, b, blk):
    n, k = x.shape
    _, m = w.shape
    grid = n // blk
    return pl.pallas_call(
        _matmul_bias_body,
        grid=(grid,),
        in_specs=[
            pl.BlockSpec((blk, k), lambda i: (i, 0)),
            pl.BlockSpec((k, m), lambda i: (0, 0)),
            pl.BlockSpec((1, m), lambda i: (0, 0)),
        ],
        out_specs=pl.BlockSpec((blk, m), lambda i: (i, 0)),
        out_shape=jax.ShapeDtypeStruct((n, m), F32),
    )(x, w, b.reshape(1, m))


def _mlp_body(h_ref, a0_ref, a1_ref, w1_ref, b1_ref, w2_ref, b2_ref, o_ref,
              *, final):
    h_in = h_ref[...] + a0_ref[...] + a1_ref[...]
    mid = jnp.maximum(
        jnp.dot(h_in, w1_ref[...], preferred_element_type=F32) + b1_ref[...], 0.0
    )
    out = jnp.dot(mid, w2_ref[...], preferred_element_type=F32) + b2_ref[...]
    if not final:
        out = jnp.maximum(out, 0.0)
    o_ref[...] = out


def _gin_mlp(h, agg2, w1, b1, w2, b2, final, blk):
    n, d = h.shape
    dh = w1.shape[1]
    nblk = n // blk
    return pl.pallas_call(
        functools.partial(_mlp_body, final=final),
        grid=(nblk,),
        in_specs=[
            pl.BlockSpec((blk, d), lambda i: (i, 0)),
            pl.BlockSpec((blk, d), lambda i: (i, 0)),
            pl.BlockSpec((blk, d), lambda i, _n=nblk: (i + _n, 0)),
            pl.BlockSpec((d, dh), lambda i: (0, 0)),
            pl.BlockSpec((1, dh), lambda i: (0, 0)),
            pl.BlockSpec((dh, d), lambda i: (0, 0)),
            pl.BlockSpec((1, d), lambda i: (0, 0)),
        ],
        out_specs=pl.BlockSpec((blk, d), lambda i: (i, 0)),
        out_shape=jax.ShapeDtypeStruct((n, d), F32),
    )(h, agg2, agg2, w1, b1.reshape(1, dh), w2, b2.reshape(1, d))


def _mish_tc(x):
    sp = jnp.maximum(x, 0.0) + jnp.log1p(jnp.exp(-jnp.abs(x)))
    return x * jnp.tanh(sp)


def _pool_body(node_ref, batch_ref, gpw_ref, gpb_ref, nfwb_ref,
               grep_ref, gout_ref, gtnp_ref, acc_ref, cnt_ref, *, ngraph, nsteps):
    i = pl.program_id(0)

    @pl.when(i == 0)
    def _():
        acc_ref[...] = jnp.zeros_like(acc_ref)
        cnt_ref[...] = jnp.zeros_like(cnt_ref)

    bblk = batch_ref[0, 0, :]
    onehot = (
        lax.broadcasted_iota(jnp.int32, (ngraph, bblk.shape[0]), 0)
        == bblk[None, :]
    ).astype(F32)
    acc_ref[...] += jnp.dot(onehot, node_ref[...], preferred_element_type=F32)
    cnt_ref[...] += jnp.sum(onehot, axis=1, keepdims=True)

    @pl.when(i == nsteps - 1)
    def _():
        grep = acc_ref[...] / jnp.maximum(cnt_ref[...], 1.0)
        grep_ref[...] = grep
        gout_ref[...] = (
            jnp.dot(grep, gpw_ref[...], preferred_element_type=F32) + gpb_ref[...]
        )
        gtnp_ref[...] = jnp.dot(grep, nfwb_ref[...], preferred_element_type=F32)


def _pool(node_rep, batch3, gp_w, gp_b, nf_w_bot, blk):
    n, d = node_rep.shape
    g = gp_w.shape[0] if gp_w.shape[0] != d else 256
    g = 256
    nsteps = n // blk
    return pl.pallas_call(
        functools.partial(_pool_body, ngraph=g, nsteps=nsteps),
        grid=(nsteps,),
        in_specs=[
            pl.BlockSpec((blk, d), lambda i: (i, 0)),
            pl.BlockSpec((1, 1, blk), lambda i: (i, 0, 0)),
            pl.BlockSpec((d, d), lambda i: (0, 0)),
            pl.BlockSpec((1, d), lambda i: (0, 0)),
            pl.BlockSpec((d, d), lambda i: (0, 0)),
        ],
        out_specs=[
            pl.BlockSpec((g, d), lambda i: (0, 0)),
            pl.BlockSpec((g, d), lambda i: (0, 0)),
            pl.BlockSpec((g, d), lambda i: (0, 0)),
        ],
        out_shape=[
            jax.ShapeDtypeStruct((g, d), F32),
            jax.ShapeDtypeStruct((g, d), F32),
            jax.ShapeDtypeStruct((g, d), F32),
        ],
        scratch_shapes=[
            pltpu.VMEM((g, d), F32),
            pltpu.VMEM((g, 1), F32),
        ],
    )(node_rep, batch3, gp_w, gp_b.reshape(1, d), nf_w_bot)


def _heads_body(node_ref, batch_ref, gtnp_ref, nfwt_ref, nfb_ref,
                efwt_ref, efwb_ref, efb_ref,
                n2_ref, se_ref, a_ref, bt_ref, *, ngraph):
    bblk = batch_ref[0, 0, :]
    onehot = (
        lax.broadcasted_iota(jnp.int32, (bblk.shape[0], ngraph), 1)
        == bblk[:, None]
    ).astype(F32)
    gtn = jnp.dot(onehot, gtnp_ref[...], preferred_element_type=F32)
    pre = (
        jnp.dot(node_ref[...], nfwt_ref[...], preferred_element_type=F32)
        + gtn + nfb_ref[...]
    )
    n2 = _mish_tc(pre)
    n2_ref[...] = n2
    a = jnp.dot(n2, efwt_ref[...], preferred_element_type=F32)
    bt = jnp.dot(n2, efwb_ref[...], preferred_element_type=F32) + efb_ref[...]
    a_ref[...] = a
    bt_ref[...] = bt
    se_ref[...] = _mish_tc(a + bt)


def _heads(node_rep, batch3, gtnp, nf_w_top, nf_b, ef_w_top, ef_w_bot, ef_b, blk):
    n, d = node_rep.shape
    g = gtnp.shape[0]
    nsteps = n // blk
    outs = pl.pallas_call(
        functools.partial(_heads_body, ngraph=g),
        grid=(nsteps,),
        in_specs=[
            pl.BlockSpec((blk, d), lambda i: (i, 0)),
            pl.BlockSpec((1, 1, blk), lambda i: (i, 0, 0)),
            pl.BlockSpec((g, d), lambda i: (0, 0)),
            pl.BlockSpec((d, d), lambda i: (0, 0)),
            pl.BlockSpec((1, d), lambda i: (0, 0)),
            pl.BlockSpec((d, d), lambda i: (0, 0)),
            pl.BlockSpec((d, d), lambda i: (0, 0)),
            pl.BlockSpec((1, d), lambda i: (0, 0)),
        ],
        out_specs=[pl.BlockSpec((blk, d), lambda i: (i, 0))] * 4,
        out_shape=[jax.ShapeDtypeStruct((n, d), F32)] * 4,
    )(node_rep, batch3, gtnp, nf_w_top, nf_b.reshape(1, d),
      ef_w_top, ef_w_bot, ef_b.reshape(1, d))
    return outs


_EBLK = 128


_MBLK = 128


def _msgpass_body(h_hbm, e_hbm, src_hbm, dst_hbm, zeros_hbm, out_hbm,
                  agg, src_v, dst_v, rows_v, e_v, gsem,
                  *, n_nodes, n_pad, n_edges, emb):
    c = lax.axis_index("c")
    s = lax.axis_index("s")
    blk_core = (n_edges // 2) // _MBLK
    bpt = blk_core // 16
    rows_per_tile = n_pad // 16

    zlo = s * rows_per_tile
    pltpu.sync_copy(zeros_hbm.at[pl.ds(zlo, rows_per_tile)],
                    agg.at[pl.ds(zlo, rows_per_tile)])
    plsc.subcore_barrier()

    base_e = c * (n_edges // 2)

    def blk_body(i, _):
        b = s + i * 16
        e0 = base_e + b * _MBLK
        pltpu.sync_copy(src_hbm.at[pl.ds(e0, _MBLK)], src_v)
        pltpu.sync_copy(dst_hbm.at[pl.ds(e0, _MBLK)], dst_v)
        cp = pltpu.async_copy(h_hbm.at[src_v], rows_v, gsem)
        pltpu.sync_copy(e_hbm.at[pl.ds(e0, _MBLK)], e_v)
        cp.wait()

        def row_body(r, _):
            for j in range(emb // 16):
                sl = pl.ds(j * 16, 16)
                e_v[r, sl] = jnp.maximum(rows_v[r, sl] + e_v[r, sl], 0.0)
            return 0

        lax.fori_loop(0, _MBLK, row_body, 0)
        pltpu.sync_copy(e_v, agg.at[dst_v], add=True)
        return 0

    lax.fori_loop(0, bpt, blk_body, 0)

    plsc.subcore_barrier()
    last_rows = n_nodes - 15 * rows_per_tile

    @pl.when(s < 15)
    def _():
        pltpu.sync_copy(agg.at[pl.ds(zlo, rows_per_tile)],
                        out_hbm.at[pl.ds(c * n_nodes + zlo, rows_per_tile)])

    @pl.when(s == 15)
    def _():
        pltpu.sync_copy(agg.at[pl.ds(zlo, last_rows)],
                        out_hbm.at[pl.ds(c * n_nodes + zlo, last_rows)])


def _msgpass(h, e, src, dst, zeros):
    n, emb = h.shape
    n_pad = zeros.shape[0]
    n_edges = e.shape[0]
    mesh = plsc.VectorSubcoreMesh(core_axis_name="c", subcore_axis_name="s")
    body = functools.partial(_msgpass_body, n_nodes=n, n_pad=n_pad,
                             n_edges=n_edges, emb=emb)
    return pl.kernel(
        body,
        out_type=jax.ShapeDtypeStruct((2 * n, emb), F32),
        mesh=mesh,
        scratch_types=[
            pltpu.VMEM_SHARED((n_pad, emb), F32),
            pltpu.VMEM((_MBLK,), jnp.int32),
            pltpu.VMEM((_MBLK,), jnp.int32),
            pltpu.VMEM((_MBLK, emb), F32),
            pltpu.VMEM((_MBLK, emb), F32),
            pltpu.SemaphoreType.DMA,
        ],
    )(h, e, src, dst, zeros)


def _edgegather_body(a_hbm, b_hbm, src_hbm, dst_hbm, ag_hbm, bg_hbm,
                     src_v, dst_v, arow_v, brow_v, sem_a, sem_b,
                     *, n_edges, emb):
    c = lax.axis_index("c")
    s = lax.axis_index("s")
    w = s * 2 + c
    nblk = n_edges // _EBLK
    ntile = (nblk - w + 31) // 32

    def blk_body(i, _):
        b = w + i * 32
        e0 = b * _EBLK
        pltpu.sync_copy(src_hbm.at[pl.ds(e0, _EBLK)], src_v)
        pltpu.sync_copy(dst_hbm.at[pl.ds(e0, _EBLK)], dst_v)
        ca = pltpu.async_copy(a_hbm.at[src_v], arow_v, sem_a)
        cb = pltpu.async_copy(b_hbm.at[dst_v], brow_v, sem_b)
        ca.wait()
        cb.wait()
        pltpu.sync_copy(arow_v, ag_hbm.at[pl.ds(e0, _EBLK)])
        pltpu.sync_copy(brow_v, bg_hbm.at[pl.ds(e0, _EBLK)])
        return 0

    lax.fori_loop(0, ntile, blk_body, 0)


def _edgegather(a, bt, src, dst):
    n, emb = a.shape
    n_edges = src.shape[0]
    mesh = plsc.VectorSubcoreMesh(core_axis_name="c", subcore_axis_name="s")
    body = functools.partial(_edgegather_body, n_edges=n_edges, emb=emb)
    return pl.kernel(
        body,
        out_type=[jax.ShapeDtypeStruct((n_edges, emb), F32)] * 2,
        mesh=mesh,
        scratch_types=[
            pltpu.VMEM((_EBLK,), jnp.int32),
            pltpu.VMEM((_EBLK,), jnp.int32),
            pltpu.VMEM((_EBLK, emb), F32),
            pltpu.VMEM((_EBLK, emb), F32),
            pltpu.SemaphoreType.DMA,
            pltpu.SemaphoreType.DMA,
        ],
    )(a, bt, src, dst)


def _edgemish_body(ag_ref, bg_ref, o_ref):
    x = ag_ref[...] + bg_ref[...]
    m = _mish_tc(x)
    o_ref[...] = 0.5 * (m[:, 0, :] + m[:, 1, :])


def _edgemish(ag, bg, n_edges, blk):
    emb = ag.shape[1]
    half = n_edges // 2
    ag2 = ag.reshape(ag.shape[0] // 2, 2, emb)
    bg2 = bg.reshape(bg.shape[0] // 2, 2, emb)
    grid = half // blk
    return pl.pallas_call(
        _edgemish_body,
        grid=(grid,),
        in_specs=[
            pl.BlockSpec((blk, 2, emb), lambda i: (i, 0, 0)),
            pl.BlockSpec((blk, 2, emb), lambda i: (i, 0, 0)),
        ],
        out_specs=pl.BlockSpec((blk, emb), lambda i: (i, 0)),
        out_shape=jax.ShapeDtypeStruct((half, emb), F32),
    )(ag2, bg2)


def kernel(x, edge_index, edge_attr, batch, params):
    n, _ = x.shape
    emb = params['atom_w'].shape[1]
    num_layers = params['edge_w'].shape[0]
    src = edge_index[0]
    dst = edge_index[1]
    n_edges = src.shape[0]
    nblk = 1000
    batch3 = batch.reshape(n // nblk, 1, nblk)
    rpt = ((n + 15) // 16 + 7) // 8 * 8
    n_pad = 16 * rpt
    zeros = jnp.zeros((n_pad, emb), F32)

    unit = 2 * 16 * 2 * _EBLK
    e_pad = ((n_edges + unit - 1) // unit) * unit
    pad = e_pad - n_edges
    ea_pad = jnp.pad(edge_attr, ((0, pad), (0, 0)))
    src_p = jnp.pad(src, (0, pad))
    pad_dst = n + (jnp.arange(pad, dtype=jnp.int32) % (n_pad - n))
    dst_mp = jnp.concatenate([dst, pad_dst])
    dst_eg = jnp.pad(dst, (0, pad))

    h = _matmul_bias(x, params['atom_w'], params['atom_b'], nblk)
    for l in range(num_layers):
        e = _matmul_bias(ea_pad, params['edge_w'][l], params['edge_b'][l], 4096)
        agg2 = _msgpass(h, e, src_p, dst_mp, zeros)
        h = _gin_mlp(h, agg2, params['mlp_w1'][l], params['mlp_b1'][l],
                     params['mlp_w2'][l], params['mlp_b2'][l],
                     final=(l == num_layers - 1), blk=nblk)

    nf_w = params['nf_w']
    ef_w = params['ef_w']
    _, graph_out, gtnp = _pool(h, batch3, params['gp_w'], params['gp_b'],
                               nf_w[emb:], nblk)
    n2, se, a, bt = _heads(h, batch3, gtnp, nf_w[:emb], params['nf_b'],
                           ef_w[:emb], ef_w[emb:], params['ef_b'], nblk)
    ag, bg = _edgegather(a, bt, src_p, dst_eg)
    edge_rep = _edgemish(ag, bg, n_edges, 2000)
    return (n2, se, edge_rep, graph_out)


# speedup vs baseline: 1.5703x; 1.5031x over previous
import functools

import jax
import jax.numpy as jnp
import numpy as np
from jax import lax
from jax.experimental import pallas as pl
from jax.experimental.pallas import tpu as pltpu
from jax.experimental.pallas import tpu_sc as plsc

F32 = jnp.float32


def _matmul_bias_body(x_ref, w_ref, b_ref, o_ref):
    o_ref[...] = (
        jnp.dot(x_ref[...], w_ref[...], preferred_element_type=F32) + b_ref[...]
    )


def _matmul_bias(x, w, b, blk):
    n, k = x.shape
    _, m = w.shape
    grid = n // blk
    return pl.pallas_call(
        _matmul_bias_body,
        grid=(grid,),
        in_specs=[
            pl.BlockSpec((blk, k), lambda i: (i, 0)),
            pl.BlockSpec((k, m), lambda i: (0, 0)),
            pl.BlockSpec((1, m), lambda i: (0, 0)),
        ],
        out_specs=pl.BlockSpec((blk, m), lambda i: (i, 0)),
        out_shape=jax.ShapeDtypeStruct((n, m), F32),
    )(x, w, b.reshape(1, m))


def _mlp_body(h_ref, a0_ref, a1_ref, w1_ref, b1_ref, w2_ref, b2_ref, o_ref,
              *, final):
    h_in = h_ref[...] + a0_ref[...] + a1_ref[...]
    mid = jnp.maximum(
        jnp.dot(h_in, w1_ref[...], preferred_element_type=F32) + b1_ref[...], 0.0
    )
    out = jnp.dot(mid, w2_ref[...], preferred_element_type=F32) + b2_ref[...]
    if not final:
        out = jnp.maximum(out, 0.0)
    o_ref[...] = out


def _gin_mlp(h, agg2, w1, b1, w2, b2, final, blk):
    n, d = h.shape
    dh = w1.shape[1]
    nblk = n // blk
    return pl.pallas_call(
        functools.partial(_mlp_body, final=final),
        grid=(nblk,),
        in_specs=[
            pl.BlockSpec((blk, d), lambda i: (i, 0)),
            pl.BlockSpec((blk, d), lambda i: (i, 0)),
            pl.BlockSpec((blk, d), lambda i, _n=nblk: (i + _n, 0)),
            pl.BlockSpec((d, dh), lambda i: (0, 0)),
            pl.BlockSpec((1, dh), lambda i: (0, 0)),
            pl.BlockSpec((dh, d), lambda i: (0, 0)),
            pl.BlockSpec((1, d), lambda i: (0, 0)),
        ],
        out_specs=pl.BlockSpec((blk, d), lambda i: (i, 0)),
        out_shape=jax.ShapeDtypeStruct((n, d), F32),
    )(h, agg2, agg2, w1, b1.reshape(1, dh), w2, b2.reshape(1, d))


def _mish_tc(x):
    sp = jnp.maximum(x, 0.0) + jnp.log1p(jnp.exp(-jnp.abs(x)))
    return x * jnp.tanh(sp)


def _pool_body(node_ref, batch_ref, gpw_ref, gpb_ref, nfwb_ref,
               grep_ref, gout_ref, gtnp_ref, acc_ref, cnt_ref, *, ngraph, nsteps):
    i = pl.program_id(0)

    @pl.when(i == 0)
    def _():
        acc_ref[...] = jnp.zeros_like(acc_ref)
        cnt_ref[...] = jnp.zeros_like(cnt_ref)

    bblk = batch_ref[0, 0, :]
    onehot = (
        lax.broadcasted_iota(jnp.int32, (ngraph, bblk.shape[0]), 0)
        == bblk[None, :]
    ).astype(F32)
    acc_ref[...] += jnp.dot(onehot, node_ref[...], preferred_element_type=F32)
    cnt_ref[...] += jnp.sum(onehot, axis=1, keepdims=True)

    @pl.when(i == nsteps - 1)
    def _():
        grep = acc_ref[...] / jnp.maximum(cnt_ref[...], 1.0)
        grep_ref[...] = grep
        gout_ref[...] = (
            jnp.dot(grep, gpw_ref[...], preferred_element_type=F32) + gpb_ref[...]
        )
        gtnp_ref[...] = jnp.dot(grep, nfwb_ref[...], preferred_element_type=F32)


def _pool(node_rep, batch3, gp_w, gp_b, nf_w_bot, blk):
    n, d = node_rep.shape
    g = gp_w.shape[0] if gp_w.shape[0] != d else 256
    g = 256
    nsteps = n // blk
    return pl.pallas_call(
        functools.partial(_pool_body, ngraph=g, nsteps=nsteps),
        grid=(nsteps,),
        in_specs=[
            pl.BlockSpec((blk, d), lambda i: (i, 0)),
            pl.BlockSpec((1, 1, blk), lambda i: (i, 0, 0)),
            pl.BlockSpec((d, d), lambda i: (0, 0)),
            pl.BlockSpec((1, d), lambda i: (0, 0)),
            pl.BlockSpec((d, d), lambda i: (0, 0)),
        ],
        out_specs=[
            pl.BlockSpec((g, d), lambda i: (0, 0)),
            pl.BlockSpec((g, d), lambda i: (0, 0)),
            pl.BlockSpec((g, d), lambda i: (0, 0)),
        ],
        out_shape=[
            jax.ShapeDtypeStruct((g, d), F32),
            jax.ShapeDtypeStruct((g, d), F32),
            jax.ShapeDtypeStruct((g, d), F32),
        ],
        scratch_shapes=[
            pltpu.VMEM((g, d), F32),
            pltpu.VMEM((g, 1), F32),
        ],
    )(node_rep, batch3, gp_w, gp_b.reshape(1, d), nf_w_bot)


def _heads_body(node_ref, batch_ref, gtnp_ref, nfwt_ref, nfb_ref,
                efwt_ref, efwb_ref, efb_ref,
                n2_ref, se_ref, a_ref, bt_ref, *, ngraph):
    bblk = batch_ref[0, 0, :]
    onehot = (
        lax.broadcasted_iota(jnp.int32, (bblk.shape[0], ngraph), 1)
        == bblk[:, None]
    ).astype(F32)
    gtn = jnp.dot(onehot, gtnp_ref[...], preferred_element_type=F32)
    pre = (
        jnp.dot(node_ref[...], nfwt_ref[...], preferred_element_type=F32)
        + gtn + nfb_ref[...]
    )
    n2 = _mish_tc(pre)
    n2_ref[...] = n2
    a = jnp.dot(n2, efwt_ref[...], preferred_element_type=F32)
    bt = jnp.dot(n2, efwb_ref[...], preferred_element_type=F32) + efb_ref[...]
    a_ref[...] = a
    bt_ref[...] = bt
    se_ref[...] = _mish_tc(a + bt)


def _heads(node_rep, batch3, gtnp, nf_w_top, nf_b, ef_w_top, ef_w_bot, ef_b, blk):
    n, d = node_rep.shape
    g = gtnp.shape[0]
    nsteps = n // blk
    outs = pl.pallas_call(
        functools.partial(_heads_body, ngraph=g),
        grid=(nsteps,),
        in_specs=[
            pl.BlockSpec((blk, d), lambda i: (i, 0)),
            pl.BlockSpec((1, 1, blk), lambda i: (i, 0, 0)),
            pl.BlockSpec((g, d), lambda i: (0, 0)),
            pl.BlockSpec((d, d), lambda i: (0, 0)),
            pl.BlockSpec((1, d), lambda i: (0, 0)),
            pl.BlockSpec((d, d), lambda i: (0, 0)),
            pl.BlockSpec((d, d), lambda i: (0, 0)),
            pl.BlockSpec((1, d), lambda i: (0, 0)),
        ],
        out_specs=[pl.BlockSpec((blk, d), lambda i: (i, 0))] * 4,
        out_shape=[jax.ShapeDtypeStruct((n, d), F32)] * 4,
    )(node_rep, batch3, gtnp, nf_w_top, nf_b.reshape(1, d),
      ef_w_top, ef_w_bot, ef_b.reshape(1, d))
    return outs


_EBLK = 128


_MBLK = 128


def _msgpass_body(h_hbm, e_hbm, src_hbm, dst_hbm, zeros_hbm, out_hbm,
                  agg, src_v, dst_v, rows_v, e_v, gsem,
                  *, n_nodes, n_pad, n_edges, emb):
    c = lax.axis_index("c")
    s = lax.axis_index("s")
    blk_core = (n_edges // 2) // _MBLK
    bpt = blk_core // 16
    rows_per_tile = n_pad // 16

    zlo = s * rows_per_tile
    pltpu.sync_copy(zeros_hbm.at[pl.ds(zlo, rows_per_tile)],
                    agg.at[pl.ds(zlo, rows_per_tile)])
    plsc.subcore_barrier()

    base_e = c * (n_edges // 2)

    def blk_body(i, _):
        b = s + i * 16
        e0 = base_e + b * _MBLK
        pltpu.sync_copy(src_hbm.at[pl.ds(e0, _MBLK)], src_v)
        pltpu.sync_copy(dst_hbm.at[pl.ds(e0, _MBLK)], dst_v)
        cp = pltpu.async_copy(h_hbm.at[src_v], rows_v, gsem)
        pltpu.sync_copy(e_hbm.at[pl.ds(e0, _MBLK)], e_v)
        cp.wait()

        def row_body(r, _):
            for j in range(emb // 16):
                sl = pl.ds(j * 16, 16)
                e_v[r, sl] = jnp.maximum(rows_v[r, sl] + e_v[r, sl], 0.0)
            return 0

        lax.fori_loop(0, _MBLK, row_body, 0)
        pltpu.sync_copy(e_v, agg.at[dst_v], add=True)
        return 0

    lax.fori_loop(0, bpt, blk_body, 0)

    plsc.subcore_barrier()
    last_rows = n_nodes - 15 * rows_per_tile

    @pl.when(s < 15)
    def _():
        pltpu.sync_copy(agg.at[pl.ds(zlo, rows_per_tile)],
                        out_hbm.at[pl.ds(c * n_nodes + zlo, rows_per_tile)])

    @pl.when(s == 15)
    def _():
        pltpu.sync_copy(agg.at[pl.ds(zlo, last_rows)],
                        out_hbm.at[pl.ds(c * n_nodes + zlo, last_rows)])


def _msgpass(h, e, src, dst, zeros):
    n, emb = h.shape
    n_pad = zeros.shape[0]
    n_edges = e.shape[0]
    mesh = plsc.VectorSubcoreMesh(core_axis_name="c", subcore_axis_name="s")
    body = functools.partial(_msgpass_body, n_nodes=n, n_pad=n_pad,
                             n_edges=n_edges, emb=emb)
    return pl.kernel(
        body,
        out_type=jax.ShapeDtypeStruct((2 * n, emb), F32),
        mesh=mesh,
        scratch_types=[
            pltpu.VMEM_SHARED((n_pad, emb), F32),
            pltpu.VMEM((_MBLK,), jnp.int32),
            pltpu.VMEM((_MBLK,), jnp.int32),
            pltpu.VMEM((_MBLK, emb), F32),
            pltpu.VMEM((_MBLK, emb), F32),
            pltpu.SemaphoreType.DMA,
        ],
    )(h, e, src, dst, zeros)


def _edgegather_body(a_hbm, b_hbm, src_hbm, dst_hbm, ag_hbm, bg_hbm,
                     src_v, dst_v, arow_v, brow_v, sem_a, sem_b,
                     *, n_edges, emb):
    c = lax.axis_index("c")
    s = lax.axis_index("s")
    w = s * 2 + c
    nblk = n_edges // _EBLK
    ntile = (nblk - w + 31) // 32

    def blk_body(i, _):
        b = w + i * 32
        e0 = b * _EBLK
        pltpu.sync_copy(src_hbm.at[pl.ds(e0, _EBLK)], src_v)
        pltpu.sync_copy(dst_hbm.at[pl.ds(e0, _EBLK)], dst_v)
        ca = pltpu.async_copy(a_hbm.at[src_v], arow_v, sem_a)
        cb = pltpu.async_copy(b_hbm.at[dst_v], brow_v, sem_b)
        ca.wait()
        cb.wait()
        pltpu.sync_copy(arow_v, ag_hbm.at[pl.ds(e0, _EBLK)])
        pltpu.sync_copy(brow_v, bg_hbm.at[pl.ds(e0, _EBLK)])
        return 0

    lax.fori_loop(0, ntile, blk_body, 0)


def _edgegather(a, bt, src, dst):
    n, emb = a.shape
    n_edges = src.shape[0]
    mesh = plsc.VectorSubcoreMesh(core_axis_name="c", subcore_axis_name="s")
    body = functools.partial(_edgegather_body, n_edges=n_edges, emb=emb)
    return pl.kernel(
        body,
        out_type=[jax.ShapeDtypeStruct((n_edges, emb), F32)] * 2,
        mesh=mesh,
        scratch_types=[
            pltpu.VMEM((_EBLK,), jnp.int32),
            pltpu.VMEM((_EBLK,), jnp.int32),
            pltpu.VMEM((_EBLK, emb), F32),
            pltpu.VMEM((_EBLK, emb), F32),
            pltpu.SemaphoreType.DMA,
            pltpu.SemaphoreType.DMA,
        ],
    )(a, bt, src, dst)


def _edgemish_body(ag_ref, bg_ref, o_ref):
    x = ag_ref[...] + bg_ref[...]
    m = _mish_tc(x)
    o_ref[...] = 0.5 * (m[:, 0, :] + m[:, 1, :])


def _edgemish(ag, bg, n_edges, blk):
    emb = ag.shape[1]
    half = n_edges // 2
    ag2 = ag.reshape(ag.shape[0] // 2, 2, emb)
    bg2 = bg.reshape(bg.shape[0] // 2, 2, emb)
    grid = half // blk
    return pl.pallas_call(
        _edgemish_body,
        grid=(grid,),
        in_specs=[
            pl.BlockSpec((blk, 2, emb), lambda i: (i, 0, 0)),
            pl.BlockSpec((blk, 2, emb), lambda i: (i, 0, 0)),
        ],
        out_specs=pl.BlockSpec((blk, emb), lambda i: (i, 0)),
        out_shape=jax.ShapeDtypeStruct((half, emb), F32),
    )(ag2, bg2)


def kernel(x, edge_index, edge_attr, batch, params):
    n, _ = x.shape
    emb = params['atom_w'].shape[1]
    num_layers = params['edge_w'].shape[0]
    src = edge_index[0]
    dst = edge_index[1]
    n_edges = src.shape[0]
    nblk = 1000
    batch3 = batch.reshape(n // nblk, 1, nblk)
    rpt = ((n + 15) // 16 + 7) // 8 * 8
    n_pad = 16 * rpt
    zeros = jnp.zeros((n_pad, emb), F32)

    unit = 2 * 16 * 2 * _EBLK
    e_pad = ((n_edges + unit - 1) // unit) * unit
    pad = e_pad - n_edges
    ea_pad = jnp.pad(edge_attr, ((0, pad), (0, 0)))
    pad_spread = jnp.arange(pad, dtype=jnp.int32) % n
    src_p = jnp.concatenate([src, pad_spread])
    pad_dst = n + (jnp.arange(pad, dtype=jnp.int32) % (n_pad - n))
    dst_mp = jnp.concatenate([dst, pad_dst])
    dst_eg = jnp.concatenate([dst, pad_spread])

    h = _matmul_bias(x, params['atom_w'], params['atom_b'], nblk)
    for l in range(num_layers):
        e = _matmul_bias(ea_pad, params['edge_w'][l], params['edge_b'][l], 4096)
        agg2 = _msgpass(h, e, src_p, dst_mp, zeros)
        h = _gin_mlp(h, agg2, params['mlp_w1'][l], params['mlp_b1'][l],
                     params['mlp_w2'][l], params['mlp_b2'][l],
                     final=(l == num_layers - 1), blk=nblk)

    nf_w = params['nf_w']
    ef_w = params['ef_w']
    _, graph_out, gtnp = _pool(h, batch3, params['gp_w'], params['gp_b'],
                               nf_w[emb:], nblk)
    n2, se, a, bt = _heads(h, batch3, gtnp, nf_w[:emb], params['nf_b'],
                           ef_w[:emb], ef_w[emb:], params['ef_b'], nblk)
    ag, bg = _edgegather(a, bt, src_p, dst_eg)
    edge_rep = _edgemish(ag, bg, n_edges, 2000)
    return (n2, se, edge_rep, graph_out)


# speedup vs baseline: 2.0849x; 1.3277x over previous
import functools

import jax
import jax.numpy as jnp
import numpy as np
from jax import lax
from jax.experimental import pallas as pl
from jax.experimental.pallas import tpu as pltpu
from jax.experimental.pallas import tpu_sc as plsc

F32 = jnp.float32


def _matmul_bias_body(x_ref, w_ref, b_ref, o_ref):
    o_ref[...] = (
        jnp.dot(x_ref[...], w_ref[...], preferred_element_type=F32) + b_ref[...]
    )


def _matmul_bias(x, w, b, blk):
    n, k = x.shape
    _, m = w.shape
    grid = n // blk
    return pl.pallas_call(
        _matmul_bias_body,
        grid=(grid,),
        in_specs=[
            pl.BlockSpec((blk, k), lambda i: (i, 0)),
            pl.BlockSpec((k, m), lambda i: (0, 0)),
            pl.BlockSpec((1, m), lambda i: (0, 0)),
        ],
        out_specs=pl.BlockSpec((blk, m), lambda i: (i, 0)),
        out_shape=jax.ShapeDtypeStruct((n, m), F32),
    )(x, w, b.reshape(1, m))


def _mlp_body(h_ref, a0_ref, a1_ref, w1_ref, b1_ref, w2_ref, b2_ref, o_ref,
              *, final):
    h_in = h_ref[...] + a0_ref[...] + a1_ref[...]
    mid = jnp.maximum(
        jnp.dot(h_in, w1_ref[...], preferred_element_type=F32) + b1_ref[...], 0.0
    )
    out = jnp.dot(mid, w2_ref[...], preferred_element_type=F32) + b2_ref[...]
    if not final:
        out = jnp.maximum(out, 0.0)
    o_ref[...] = out


def _gin_mlp(h, agg2, w1, b1, w2, b2, final, blk):
    n, d = h.shape
    dh = w1.shape[1]
    nblk = n // blk
    return pl.pallas_call(
        functools.partial(_mlp_body, final=final),
        grid=(nblk,),
        in_specs=[
            pl.BlockSpec((blk, d), lambda i: (i, 0)),
            pl.BlockSpec((blk, d), lambda i: (i, 0)),
            pl.BlockSpec((blk, d), lambda i, _n=nblk: (i + _n, 0)),
            pl.BlockSpec((d, dh), lambda i: (0, 0)),
            pl.BlockSpec((1, dh), lambda i: (0, 0)),
            pl.BlockSpec((dh, d), lambda i: (0, 0)),
            pl.BlockSpec((1, d), lambda i: (0, 0)),
        ],
        out_specs=pl.BlockSpec((blk, d), lambda i: (i, 0)),
        out_shape=jax.ShapeDtypeStruct((n, d), F32),
    )(h, agg2, agg2, w1, b1.reshape(1, dh), w2, b2.reshape(1, d))


def _mish_tc(x):
    sp = jnp.maximum(x, 0.0) + jnp.log1p(jnp.exp(-jnp.abs(x)))
    return x * jnp.tanh(sp)


def _pool_body(node_ref, batch_ref, gpw_ref, gpb_ref, nfwb_ref,
               grep_ref, gout_ref, gtnp_ref, acc_ref, cnt_ref, *, ngraph, nsteps):
    i = pl.program_id(0)

    @pl.when(i == 0)
    def _():
        acc_ref[...] = jnp.zeros_like(acc_ref)
        cnt_ref[...] = jnp.zeros_like(cnt_ref)

    bblk = batch_ref[0, 0, :]
    onehot = (
        lax.broadcasted_iota(jnp.int32, (ngraph, bblk.shape[0]), 0)
        == bblk[None, :]
    ).astype(F32)
    acc_ref[...] += jnp.dot(onehot, node_ref[...], preferred_element_type=F32)
    cnt_ref[...] += jnp.sum(onehot, axis=1, keepdims=True)

    @pl.when(i == nsteps - 1)
    def _():
        grep = acc_ref[...] / jnp.maximum(cnt_ref[...], 1.0)
        grep_ref[...] = grep
        gout_ref[...] = (
            jnp.dot(grep, gpw_ref[...], preferred_element_type=F32) + gpb_ref[...]
        )
        gtnp_ref[...] = jnp.dot(grep, nfwb_ref[...], preferred_element_type=F32)


def _pool(node_rep, batch3, gp_w, gp_b, nf_w_bot, blk):
    n, d = node_rep.shape
    g = gp_w.shape[0] if gp_w.shape[0] != d else 256
    g = 256
    nsteps = n // blk
    return pl.pallas_call(
        functools.partial(_pool_body, ngraph=g, nsteps=nsteps),
        grid=(nsteps,),
        in_specs=[
            pl.BlockSpec((blk, d), lambda i: (i, 0)),
            pl.BlockSpec((1, 1, blk), lambda i: (i, 0, 0)),
            pl.BlockSpec((d, d), lambda i: (0, 0)),
            pl.BlockSpec((1, d), lambda i: (0, 0)),
            pl.BlockSpec((d, d), lambda i: (0, 0)),
        ],
        out_specs=[
            pl.BlockSpec((g, d), lambda i: (0, 0)),
            pl.BlockSpec((g, d), lambda i: (0, 0)),
            pl.BlockSpec((g, d), lambda i: (0, 0)),
        ],
        out_shape=[
            jax.ShapeDtypeStruct((g, d), F32),
            jax.ShapeDtypeStruct((g, d), F32),
            jax.ShapeDtypeStruct((g, d), F32),
        ],
        scratch_shapes=[
            pltpu.VMEM((g, d), F32),
            pltpu.VMEM((g, 1), F32),
        ],
    )(node_rep, batch3, gp_w, gp_b.reshape(1, d), nf_w_bot)


def _heads_body(node_ref, batch_ref, gtnp_ref, nfwt_ref, nfb_ref,
                efwt_ref, efwb_ref, efb_ref,
                n2_ref, se_ref, a_ref, bt_ref, *, ngraph):
    bblk = batch_ref[0, 0, :]
    onehot = (
        lax.broadcasted_iota(jnp.int32, (bblk.shape[0], ngraph), 1)
        == bblk[:, None]
    ).astype(F32)
    gtn = jnp.dot(onehot, gtnp_ref[...], preferred_element_type=F32)
    pre = (
        jnp.dot(node_ref[...], nfwt_ref[...], preferred_element_type=F32)
        + gtn + nfb_ref[...]
    )
    n2 = _mish_tc(pre)
    n2_ref[...] = n2
    a = jnp.dot(n2, efwt_ref[...], preferred_element_type=F32)
    bt = jnp.dot(n2, efwb_ref[...], preferred_element_type=F32) + efb_ref[...]
    a_ref[...] = a
    bt_ref[...] = bt
    se_ref[...] = _mish_tc(a + bt)


def _heads(node_rep, batch3, gtnp, nf_w_top, nf_b, ef_w_top, ef_w_bot, ef_b, blk):
    n, d = node_rep.shape
    g = gtnp.shape[0]
    nsteps = n // blk
    outs = pl.pallas_call(
        functools.partial(_heads_body, ngraph=g),
        grid=(nsteps,),
        in_specs=[
            pl.BlockSpec((blk, d), lambda i: (i, 0)),
            pl.BlockSpec((1, 1, blk), lambda i: (i, 0, 0)),
            pl.BlockSpec((g, d), lambda i: (0, 0)),
            pl.BlockSpec((d, d), lambda i: (0, 0)),
            pl.BlockSpec((1, d), lambda i: (0, 0)),
            pl.BlockSpec((d, d), lambda i: (0, 0)),
            pl.BlockSpec((d, d), lambda i: (0, 0)),
            pl.BlockSpec((1, d), lambda i: (0, 0)),
        ],
        out_specs=[pl.BlockSpec((blk, d), lambda i: (i, 0))] * 4,
        out_shape=[jax.ShapeDtypeStruct((n, d), F32)] * 4,
    )(node_rep, batch3, gtnp, nf_w_top, nf_b.reshape(1, d),
      ef_w_top, ef_w_bot, ef_b.reshape(1, d))
    return outs


_EBLK = 128


_MBLK = 64


def _msgpass_body(h_hbm, e_hbm, src_hbm, dst_hbm, zeros_hbm, out_hbm,
                  agg,
                  src_v0, src_v1, dst_v0, dst_v1, dst_v2, dst_v3,
                  rows_v0, rows_v1, e_v0, e_v1,
                  gsem0, gsem1, esem0, esem1,
                  issem0, issem1, idsem0, idsem1,
                  *, n_nodes, n_pad, n_edges, emb):
    c = lax.axis_index("c")
    s = lax.axis_index("s")
    blk_core = (n_edges // 2) // _MBLK
    bpt = blk_core // 16
    rows_per_tile = n_pad // 16

    src_v = (src_v0, src_v1)
    dst_v = (dst_v0, dst_v1, dst_v2, dst_v3)
    rows_v = (rows_v0, rows_v1)
    e_v = (e_v0, e_v1)
    gsem = (gsem0, gsem1)
    esem = (esem0, esem1)
    issem = (issem0, issem1)
    idsem = (idsem0, idsem1)

    zlo = s * rows_per_tile
    pltpu.sync_copy(zeros_hbm.at[pl.ds(zlo, rows_per_tile)],
                    agg.at[pl.ds(zlo, rows_per_tile)])
    plsc.subcore_barrier()

    base_e = c * (n_edges // 2)

    def eoff(i):
        return base_e + (s + i * 16) * _MBLK

    def issue_idx(i, k, k4):
        e0 = eoff(i)
        pltpu.async_copy(src_hbm.at[pl.ds(e0, _MBLK)], src_v[k], issem[k])
        pltpu.async_copy(dst_hbm.at[pl.ds(e0, _MBLK)], dst_v[k4], idsem[k])

    def wait_idx(i, k, k4):
        e0 = eoff(i)
        pltpu.make_async_copy(src_hbm.at[pl.ds(e0, _MBLK)], src_v[k],
                              issem[k]).wait()
        pltpu.make_async_copy(dst_hbm.at[pl.ds(e0, _MBLK)], dst_v[k4],
                              idsem[k]).wait()

    def issue_loads(i, k):
        e0 = eoff(i)
        pltpu.async_copy(h_hbm.at[src_v[k]], rows_v[k], gsem[k])
        pltpu.async_copy(e_hbm.at[pl.ds(e0, _MBLK)], e_v[k], esem[k])

    def wait_loads(i, k):
        e0 = eoff(i)
        pltpu.make_async_copy(h_hbm.at[src_v[k]], rows_v[k], gsem[k]).wait()
        pltpu.make_async_copy(e_hbm.at[pl.ds(e0, _MBLK)], e_v[k],
                              esem[k]).wait()

    def relu_block(k):
        rv, ev = rows_v[k], e_v[k]

        def row_body(r, _):
            for j in range(emb // 16):
                sl = pl.ds(j * 16, 16)
                ev[r, sl] = jnp.maximum(rv[r, sl] + ev[r, sl], 0.0)
            return 0

        lax.fori_loop(0, _MBLK, row_body, 0)

    def step(i, k, k4, last):
        k4n = (k4 + 2) % 4
        wait_loads(i, k)
        if not last:
            issue_idx(i + 2, k, k4n)
        relu_block(k)
        pltpu.sync_copy(e_v[k], agg.at[dst_v[k4]], add=True)
        if not last:
            wait_idx(i + 2, k, k4n)
            issue_loads(i + 2, k)

    issue_idx(0, 0, 0)
    issue_idx(1, 1, 1)
    wait_idx(0, 0, 0)
    issue_loads(0, 0)
    wait_idx(1, 1, 1)
    issue_loads(1, 1)

    step(0, 0, 0, False)
    step(1, 1, 1, False)
    step(2, 0, 2, False)
    step(3, 1, 3, False)

    def quad_body(i4, _):
        i = 4 * i4
        step(i, 0, 0, False)
        step(i + 1, 1, 1, False)
        step(i + 2, 0, 2, False)
        step(i + 3, 1, 3, False)
        return 0

    lax.fori_loop(1, bpt // 4 - 1, quad_body, 0)

    i = bpt - 4
    step(i, 0, 0, False)
    step(i + 1, 1, 1, False)
    step(i + 2, 0, 2, True)
    step(i + 3, 1, 3, True)

    plsc.subcore_barrier()
    last_rows = n_nodes - 15 * rows_per_tile

    @pl.when(s < 15)
    def _():
        pltpu.sync_copy(agg.at[pl.ds(zlo, rows_per_tile)],
                        out_hbm.at[pl.ds(c * n_nodes + zlo, rows_per_tile)])

    @pl.when(s == 15)
    def _():
        pltpu.sync_copy(agg.at[pl.ds(zlo, last_rows)],
                        out_hbm.at[pl.ds(c * n_nodes + zlo, last_rows)])


def _msgpass(h, e, src, dst, zeros):
    n, emb = h.shape
    n_pad = zeros.shape[0]
    n_edges = e.shape[0]
    mesh = plsc.VectorSubcoreMesh(core_axis_name="c", subcore_axis_name="s")
    body = functools.partial(_msgpass_body, n_nodes=n, n_pad=n_pad,
                             n_edges=n_edges, emb=emb)
    return pl.kernel(
        body,
        out_type=jax.ShapeDtypeStruct((2 * n, emb), F32),
        mesh=mesh,
        scratch_types=[
            pltpu.VMEM_SHARED((n_pad, emb), F32),
            pltpu.VMEM((_MBLK,), jnp.int32), pltpu.VMEM((_MBLK,), jnp.int32),
            pltpu.VMEM((_MBLK,), jnp.int32), pltpu.VMEM((_MBLK,), jnp.int32),
            pltpu.VMEM((_MBLK,), jnp.int32), pltpu.VMEM((_MBLK,), jnp.int32),
            pltpu.VMEM((_MBLK, emb), F32), pltpu.VMEM((_MBLK, emb), F32),
            pltpu.VMEM((_MBLK, emb), F32), pltpu.VMEM((_MBLK, emb), F32),
            pltpu.SemaphoreType.DMA, pltpu.SemaphoreType.DMA,
            pltpu.SemaphoreType.DMA, pltpu.SemaphoreType.DMA,
            pltpu.SemaphoreType.DMA, pltpu.SemaphoreType.DMA,
            pltpu.SemaphoreType.DMA, pltpu.SemaphoreType.DMA,
        ],
    )(h, e, src, dst, zeros)


def _edgegather_body(a_hbm, b_hbm, src_hbm, dst_hbm, ag_hbm, bg_hbm,
                     src_v, dst_v, arow_v, brow_v, sem_a, sem_b,
                     *, n_edges, emb):
    c = lax.axis_index("c")
    s = lax.axis_index("s")
    w = s * 2 + c
    nblk = n_edges // _EBLK
    ntile = (nblk - w + 31) // 32

    def blk_body(i, _):
        b = w + i * 32
        e0 = b * _EBLK
        pltpu.sync_copy(src_hbm.at[pl.ds(e0, _EBLK)], src_v)
        pltpu.sync_copy(dst_hbm.at[pl.ds(e0, _EBLK)], dst_v)
        ca = pltpu.async_copy(a_hbm.at[src_v], arow_v, sem_a)
        cb = pltpu.async_copy(b_hbm.at[dst_v], brow_v, sem_b)
        ca.wait()
        cb.wait()
        pltpu.sync_copy(arow_v, ag_hbm.at[pl.ds(e0, _EBLK)])
        pltpu.sync_copy(brow_v, bg_hbm.at[pl.ds(e0, _EBLK)])
        return 0

    lax.fori_loop(0, ntile, blk_body, 0)


def _edgegather(a, bt, src, dst):
    n, emb = a.shape
    n_edges = src.shape[0]
    mesh = plsc.VectorSubcoreMesh(core_axis_name="c", subcore_axis_name="s")
    body = functools.partial(_edgegather_body, n_edges=n_edges, emb=emb)
    return pl.kernel(
        body,
        out_type=[jax.ShapeDtypeStruct((n_edges, emb), F32)] * 2,
        mesh=mesh,
        scratch_types=[
            pltpu.VMEM((_EBLK,), jnp.int32),
            pltpu.VMEM((_EBLK,), jnp.int32),
            pltpu.VMEM((_EBLK, emb), F32),
            pltpu.VMEM((_EBLK, emb), F32),
            pltpu.SemaphoreType.DMA,
            pltpu.SemaphoreType.DMA,
        ],
    )(a, bt, src, dst)


def _edgemish_body(ag_ref, bg_ref, o_ref):
    x = ag_ref[...] + bg_ref[...]
    m = _mish_tc(x)
    o_ref[...] = 0.5 * (m[:, 0, :] + m[:, 1, :])


def _edgemish(ag, bg, n_edges, blk):
    emb = ag.shape[1]
    half = n_edges // 2
    ag2 = ag.reshape(ag.shape[0] // 2, 2, emb)
    bg2 = bg.reshape(bg.shape[0] // 2, 2, emb)
    grid = half // blk
    return pl.pallas_call(
        _edgemish_body,
        grid=(grid,),
        in_specs=[
            pl.BlockSpec((blk, 2, emb), lambda i: (i, 0, 0)),
            pl.BlockSpec((blk, 2, emb), lambda i: (i, 0, 0)),
        ],
        out_specs=pl.BlockSpec((blk, emb), lambda i: (i, 0)),
        out_shape=jax.ShapeDtypeStruct((half, emb), F32),
    )(ag2, bg2)


def kernel(x, edge_index, edge_attr, batch, params):
    n, _ = x.shape
    emb = params['atom_w'].shape[1]
    num_layers = params['edge_w'].shape[0]
    src = edge_index[0]
    dst = edge_index[1]
    n_edges = src.shape[0]
    nblk = 1000
    batch3 = batch.reshape(n // nblk, 1, nblk)
    rpt = ((n + 15) // 16 + 7) // 8 * 8
    n_pad = 16 * rpt
    zeros = jnp.zeros((n_pad, emb), F32)

    unit = 2 * 16 * 2 * _EBLK
    e_pad = ((n_edges + unit - 1) // unit) * unit
    pad = e_pad - n_edges
    ea_pad = jnp.pad(edge_attr, ((0, pad), (0, 0)))
    pad_spread = jnp.arange(pad, dtype=jnp.int32) % n
    src_p = jnp.concatenate([src, pad_spread])
    pad_dst = n + (jnp.arange(pad, dtype=jnp.int32) % (n_pad - n))
    dst_mp = jnp.concatenate([dst, pad_dst])
    dst_eg = jnp.concatenate([dst, pad_spread])

    h = _matmul_bias(x, params['atom_w'], params['atom_b'], nblk)
    for l in range(num_layers):
        e = _matmul_bias(ea_pad, params['edge_w'][l], params['edge_b'][l], 4096)
        agg2 = _msgpass(h, e, src_p, dst_mp, zeros)
        h = _gin_mlp(h, agg2, params['mlp_w1'][l], params['mlp_b1'][l],
                     params['mlp_w2'][l], params['mlp_b2'][l],
                     final=(l == num_layers - 1), blk=nblk)

    nf_w = params['nf_w']
    ef_w = params['ef_w']
    _, graph_out, gtnp = _pool(h, batch3, params['gp_w'], params['gp_b'],
                               nf_w[emb:], nblk)
    n2, se, a, bt = _heads(h, batch3, gtnp, nf_w[:emb], params['nf_b'],
                           ef_w[:emb], ef_w[emb:], params['ef_b'], nblk)
    ag, bg = _edgegather(a, bt, src_p, dst_eg)
    edge_rep = _edgemish(ag, bg, n_edges, 2000)
    return (n2, se, edge_rep, graph_out)
